# baseline (speedup 1.0000x reference)
import functools

import jax
import jax.numpy as jnp
from jax import lax
from jax.experimental import pallas as pl
from jax.experimental.pallas import tpu as pltpu
from jax.experimental.pallas import tpu_sc as plsc

N = 1048576
NC = 2
NS = 16
NW = NC * NS
L = 16
CH = N // NW
BLK = 8192

_mesh = plsc.VectorSubcoreMesh(
    core_axis_name="c", subcore_axis_name="s", num_cores=NC, num_subcores=NS
)


@functools.partial(
    pl.kernel,
    out_type=(
        jax.ShapeDtypeStruct((N,), jnp.float32),
        jax.ShapeDtypeStruct((N,), jnp.float32),
        jax.ShapeDtypeStruct((NW, L), jnp.int32),
    ),
    mesh=_mesh,
    scratch_types=[
        pltpu.VMEM((BLK,), jnp.float32),
        pltpu.VMEM((BLK,), jnp.int32),
        pltpu.VMEM((BLK,), jnp.float32),
        pltpu.VMEM((BLK,), jnp.float32),
        pltpu.VMEM((L,), jnp.int32),
    ],
)
def _mask_split(y_hbm, s_hbm, out0_hbm, out1_hbm, cnt_hbm, yv, sv, o0v, o1v, cv):
    wid = lax.axis_index("s") * NC + lax.axis_index("c")
    base = wid * CH
    inf16 = jnp.full((L,), jnp.inf, dtype=jnp.float32)

    def blk(b, cnt):
        off = base + b * BLK
        pltpu.sync_copy(y_hbm.at[pl.ds(off, BLK)], yv)
        pltpu.sync_copy(s_hbm.at[pl.ds(off, BLK)], sv)

        def body(k, c):
            y16 = yv[pl.ds(k * L, L)]
            s16 = sv[pl.ds(k * L, L)]
            m0 = s16 == 0
            o0v[pl.ds(k * L, L)] = jnp.where(m0, y16, inf16)
            o1v[pl.ds(k * L, L)] = jnp.where(m0, inf16, y16)
            return c + jnp.where(m0, 1, 0).astype(jnp.int32)

        cnt = lax.fori_loop(0, BLK // L, body, cnt)
        pltpu.sync_copy(o0v, out0_hbm.at[pl.ds(off, BLK)])
        pltpu.sync_copy(o1v, out1_hbm.at[pl.ds(off, BLK)])
        return cnt

    cnt = lax.fori_loop(0, CH // BLK, blk, jnp.zeros((L,), jnp.int32))
    cv[...] = cnt
    pltpu.sync_copy(cv, cnt_hbm.at[wid])


@functools.partial(
    pl.kernel,
    out_type=jax.ShapeDtypeStruct((NW, L), jnp.float32),
    mesh=_mesh,
    scratch_types=[
        pltpu.VMEM((BLK,), jnp.float32),
        pltpu.VMEM((BLK,), jnp.float32),
        pltpu.VMEM((BLK,), jnp.float32),
        pltpu.VMEM((BLK,), jnp.float32),
        pltpu.VMEM((L,), jnp.float32),
        pltpu.VMEM((L,), jnp.float32),
        pltpu.VMEM((L,), jnp.int32),
        pltpu.VMEM((L,), jnp.int32),
        pltpu.VMEM((L,), jnp.int32),
        pltpu.VMEM((L,), jnp.float32),
    ],
)
def _lerp_diff(
    a0_hbm, b0_hbm, a1_hbm, b1_hbm, r0_hbm, r1_hbm, l0_hbm, l1_hbm, m_hbm,
    out_hbm,
    a0v, b0v, a1v, b1v, r0v, r1v, l0v, l1v, mv, accv,
):
    wid = lax.axis_index("s") * NC + lax.axis_index("c")
    base = wid * CH

    pltpu.sync_copy(r0_hbm, r0v)
    pltpu.sync_copy(r1_hbm, r1v)
    pltpu.sync_copy(l0_hbm, l0v)
    pltpu.sync_copy(l1_hbm, l1v)
    pltpu.sync_copy(m_hbm, mv)
    r0 = r0v[...]
    r1 = r1v[...]
    lnm1_0 = l0v[...]
    lnm1_1 = l1v[...]
    m = mv[...]

    iota = lax.iota(jnp.int32, L)
    zero = jnp.zeros((L,), jnp.int32)

    def blk(b, acc0):
        off = base + b * BLK
        pltpu.sync_copy(a0_hbm.at[pl.ds(off, BLK)], a0v)
        pltpu.sync_copy(b0_hbm.at[pl.ds(off, BLK)], b0v)
        pltpu.sync_copy(a1_hbm.at[pl.ds(off, BLK)], a1v)
        pltpu.sync_copy(b1_hbm.at[pl.ds(off, BLK)], b1v)

        def body(k, acc):
            i16 = off + k * L + iota
            fi = i16.astype(jnp.float32)
            pos0 = fi * r0
            pos1 = fi * r1
            lo0 = jnp.minimum(jnp.maximum(pos0.astype(jnp.int32), zero), lnm1_0)
            lo1 = jnp.minimum(jnp.maximum(pos1.astype(jnp.int32), zero), lnm1_1)
            w0 = pos0 - lo0.astype(jnp.float32)
            w1 = pos1 - lo1.astype(jnp.float32)
            a0 = a0v[pl.ds(k * L, L)]
            b0 = b0v[pl.ds(k * L, L)]
            a1 = a1v[pl.ds(k * L, L)]
            b1 = b1v[pl.ds(k * L, L)]
            v0 = a0 * (1.0 - w0) + b0 * w0
            v1 = a1 * (1.0 - w1) + b1 * w1
            d = jnp.abs(v0 - v1)
            d = jnp.where(i16 < m, d, jnp.float32(0.0))
            return acc + d

        return lax.fori_loop(0, BLK // L, body, acc0)

    acc = lax.fori_loop(0, CH // BLK, blk, jnp.zeros((L,), jnp.float32))
    accv[...] = acc
    pltpu.sync_copy(accv, out_hbm.at[wid])


def kernel(y_pred, s, y_gt, pct_a, pct_b):
    s = s.astype(jnp.int32)
    masked0, masked1, counts = _mask_split(y_pred, s)
    n0 = jnp.sum(counts).astype(jnp.int32)
    n1 = jnp.int32(N) - n0

    _, sorted_all = lax.sort((s, y_pred), num_keys=2)

    pa = jnp.asarray(pct_a, jnp.int32)
    pb = jnp.asarray(pct_b, jnp.int32)
    len0 = pb * n0 - pa * n0
    len1 = pb * n1 - pa * n1
    off0 = pa * n0
    off1 = pa * n1
    m = jnp.maximum(len0, len1)
    denom = jnp.maximum(m - 1, 1).astype(jnp.float32)
    r0 = (len0 - 1).astype(jnp.float32) / denom
    r1 = (len1 - 1).astype(jnp.float32) / denom

    ar = jnp.arange(N, dtype=jnp.int32)
    fi = ar.astype(jnp.float32)
    lo0 = jnp.clip(jnp.floor(fi * r0).astype(jnp.int32), 0, len0 - 1)
    lo1 = jnp.clip(jnp.floor(fi * r1).astype(jnp.int32), 0, len1 - 1)
    hi0 = jnp.clip(lo0 + 1, 0, len0 - 1)
    hi1 = jnp.clip(lo1 + 1, 0, len1 - 1)
    inf = jnp.float32(jnp.inf)
    a0 = jnp.where(len0 > 0, sorted_all[jnp.clip(off0 + lo0, 0, N - 1)], inf)
    b0 = jnp.where(len0 > 0, sorted_all[jnp.clip(off0 + hi0, 0, N - 1)], inf)
    a1 = jnp.where(len1 > 0, sorted_all[jnp.clip(n0 + off1 + lo1, 0, N - 1)], inf)
    b1 = jnp.where(len1 > 0, sorted_all[jnp.clip(n0 + off1 + hi1, 0, N - 1)], inf)

    def splat_f(x):
        return jnp.full((L,), 1.0, jnp.float32) * x.astype(jnp.float32)

    def splat_i(x):
        return jnp.full((L,), 1, jnp.int32) * x.astype(jnp.int32)

    partials = _lerp_diff(
        a0, b0, a1, b1,
        splat_f(r0), splat_f(r1),
        splat_i(len0 - 1), splat_i(len1 - 1), splat_i(m),
    )
    reg_loss = jnp.sum(partials)

    z = jnp.zeros((1,), dtype=jnp.float32)
    return (reg_loss, z, z, z)


# speedup vs baseline: 1.0116x; 1.0116x over previous
import functools

import jax
import jax.numpy as jnp
from jax import lax
from jax.experimental import pallas as pl
from jax.experimental.pallas import tpu as pltpu
from jax.experimental.pallas import tpu_sc as plsc

N = 1048576
NC = 2
NS = 16
NW = NC * NS
L = 16
CH = N // NW
BLK = 8192

_mesh = plsc.VectorSubcoreMesh(
    core_axis_name="c", subcore_axis_name="s", num_cores=NC, num_subcores=NS
)


@functools.partial(
    pl.kernel,
    out_type=(
        jax.ShapeDtypeStruct((N,), jnp.float32),
        jax.ShapeDtypeStruct((N,), jnp.float32),
        jax.ShapeDtypeStruct((NW, L), jnp.int32),
    ),
    mesh=_mesh,
    scratch_types=[
        pltpu.VMEM((BLK,), jnp.float32),
        pltpu.VMEM((BLK,), jnp.int32),
        pltpu.VMEM((BLK,), jnp.float32),
        pltpu.VMEM((BLK,), jnp.float32),
        pltpu.VMEM((L,), jnp.int32),
    ],
)
def _mask_split(y_hbm, s_hbm, out0_hbm, out1_hbm, cnt_hbm, yv, sv, o0v, o1v, cv):
    wid = lax.axis_index("s") * NC + lax.axis_index("c")
    base = wid * CH
    inf16 = jnp.full((L,), jnp.inf, dtype=jnp.float32)

    def blk(b, cnt):
        off = base + b * BLK
        pltpu.sync_copy(y_hbm.at[pl.ds(off, BLK)], yv)
        pltpu.sync_copy(s_hbm.at[pl.ds(off, BLK)], sv)

        def body(k, c):
            y16 = yv[pl.ds(k * L, L)]
            s16 = sv[pl.ds(k * L, L)]
            m0 = s16 == 0
            o0v[pl.ds(k * L, L)] = jnp.where(m0, y16, inf16)
            o1v[pl.ds(k * L, L)] = jnp.where(m0, inf16, y16)
            return c + jnp.where(m0, 1, 0).astype(jnp.int32)

        cnt = lax.fori_loop(0, BLK // L, body, cnt)
        pltpu.sync_copy(o0v, out0_hbm.at[pl.ds(off, BLK)])
        pltpu.sync_copy(o1v, out1_hbm.at[pl.ds(off, BLK)])
        return cnt

    cnt = lax.fori_loop(0, CH // BLK, blk, jnp.zeros((L,), jnp.int32))
    cv[...] = cnt
    pltpu.sync_copy(cv, cnt_hbm.at[wid])


@functools.partial(
    pl.kernel,
    out_type=jax.ShapeDtypeStruct((NW, L), jnp.float32),
    mesh=_mesh,
    scratch_types=[
        pltpu.VMEM((BLK,), jnp.float32),
        pltpu.VMEM((BLK,), jnp.float32),
        pltpu.VMEM((BLK,), jnp.float32),
        pltpu.VMEM((BLK,), jnp.float32),
        pltpu.VMEM((L,), jnp.float32),
        pltpu.VMEM((L,), jnp.float32),
        pltpu.VMEM((L,), jnp.int32),
        pltpu.VMEM((L,), jnp.int32),
        pltpu.VMEM((L,), jnp.int32),
        pltpu.VMEM((L,), jnp.float32),
    ],
)
def _lerp_diff(
    a0_hbm, b0_hbm, a1_hbm, b1_hbm, r0_hbm, r1_hbm, l0_hbm, l1_hbm, m_hbm,
    out_hbm,
    a0v, b0v, a1v, b1v, r0v, r1v, l0v, l1v, mv, accv,
):
    wid = lax.axis_index("s") * NC + lax.axis_index("c")
    base = wid * CH

    pltpu.sync_copy(r0_hbm, r0v)
    pltpu.sync_copy(r1_hbm, r1v)
    pltpu.sync_copy(l0_hbm, l0v)
    pltpu.sync_copy(l1_hbm, l1v)
    pltpu.sync_copy(m_hbm, mv)
    r0 = r0v[...]
    r1 = r1v[...]
    lnm1_0 = l0v[...]
    lnm1_1 = l1v[...]
    m = mv[...]

    iota = lax.iota(jnp.int32, L)
    zero = jnp.zeros((L,), jnp.int32)

    def blk(b, acc0):
        off = base + b * BLK
        pltpu.sync_copy(a0_hbm.at[pl.ds(off, BLK)], a0v)
        pltpu.sync_copy(b0_hbm.at[pl.ds(off, BLK)], b0v)
        pltpu.sync_copy(a1_hbm.at[pl.ds(off, BLK)], a1v)
        pltpu.sync_copy(b1_hbm.at[pl.ds(off, BLK)], b1v)

        def body(k, acc):
            i16 = off + k * L + iota
            fi = i16.astype(jnp.float32)
            pos0 = fi * r0
            pos1 = fi * r1
            lo0 = jnp.minimum(jnp.maximum(pos0.astype(jnp.int32), zero), lnm1_0)
            lo1 = jnp.minimum(jnp.maximum(pos1.astype(jnp.int32), zero), lnm1_1)
            w0 = pos0 - lo0.astype(jnp.float32)
            w1 = pos1 - lo1.astype(jnp.float32)
            a0 = a0v[pl.ds(k * L, L)]
            b0 = b0v[pl.ds(k * L, L)]
            a1 = a1v[pl.ds(k * L, L)]
            b1 = b1v[pl.ds(k * L, L)]
            v0 = a0 * (1.0 - w0) + b0 * w0
            v1 = a1 * (1.0 - w1) + b1 * w1
            d = jnp.abs(v0 - v1)
            d = jnp.where(i16 < m, d, jnp.float32(0.0))
            return acc + d

        return lax.fori_loop(0, BLK // L, body, acc0)

    acc = lax.fori_loop(0, CH // BLK, blk, jnp.zeros((L,), jnp.float32))
    accv[...] = acc
    pltpu.sync_copy(accv, out_hbm.at[wid])


def kernel(y_pred, s, y_gt, pct_a, pct_b):
    s = s.astype(jnp.int32)
    masked0, masked1, counts = _mask_split(y_pred, s)
    n0 = jnp.sum(counts).astype(jnp.int32)
    n1 = jnp.int32(N) - n0

    sorted0 = jnp.sort(masked0)
    sorted1 = jnp.sort(masked1)

    pa = jnp.asarray(pct_a, jnp.int32)
    pb = jnp.asarray(pct_b, jnp.int32)
    len0 = pb * n0 - pa * n0
    len1 = pb * n1 - pa * n1
    off0 = pa * n0
    off1 = pa * n1
    m = jnp.maximum(len0, len1)
    denom = jnp.maximum(m - 1, 1).astype(jnp.float32)
    r0 = (len0 - 1).astype(jnp.float32) / denom
    r1 = (len1 - 1).astype(jnp.float32) / denom

    ar = jnp.arange(N, dtype=jnp.int32)
    fi = ar.astype(jnp.float32)
    lo0 = jnp.clip(jnp.floor(fi * r0).astype(jnp.int32), 0, len0 - 1)
    lo1 = jnp.clip(jnp.floor(fi * r1).astype(jnp.int32), 0, len1 - 1)
    hi0 = jnp.clip(lo0 + 1, 0, len0 - 1)
    hi1 = jnp.clip(lo1 + 1, 0, len1 - 1)
    a0 = sorted0[jnp.clip(off0 + lo0, 0, N - 1)]
    b0 = sorted0[jnp.clip(off0 + hi0, 0, N - 1)]
    a1 = sorted1[jnp.clip(off1 + lo1, 0, N - 1)]
    b1 = sorted1[jnp.clip(off1 + hi1, 0, N - 1)]

    def splat_f(x):
        return jnp.full((L,), 1.0, jnp.float32) * x.astype(jnp.float32)

    def splat_i(x):
        return jnp.full((L,), 1, jnp.int32) * x.astype(jnp.int32)

    partials = _lerp_diff(
        a0, b0, a1, b1,
        splat_f(r0), splat_f(r1),
        splat_i(len0 - 1), splat_i(len1 - 1), splat_i(m),
    )
    reg_loss = jnp.sum(partials)

    z = jnp.zeros((1,), dtype=jnp.float32)
    return (reg_loss, z, z, z)
